# Initial kernel scaffold; baseline (speedup 1.0000x reference)
import jax
import jax.numpy as jnp
from jax.experimental import pallas as pl


def kernel(token_ids, context_features, special_table, W_cls, b_cls, g_cls, beta_cls, W_ctx, b_ctx, g_ctx, beta_ctx):
    raise NotImplementedError("write your pallas kernel here")


# speedup vs baseline: 2.2418x; 2.2418x over previous
import jax
import jax.numpy as jnp
from jax.experimental import pallas as pl
from jax.experimental.pallas import tpu as pltpu

_SPECIAL_OFFSET = 20
_NUM_SPECIAL = 8
_D = 256
_NCTX = 16
_EPS = 1e-5


def _body(ids_ref, cf_ref, tab_ref, w_ref, bcls_ref, gcls_ref, pcls_ref,
          bctx_ref, gctx_ref, pctx_ref, out_ref):
    ids = ids_ref[0]
    cf = cf_ref[...]
    y = jnp.dot(cf, w_ref[...], preferred_element_type=jnp.float32)
    ycls = y[:, :_D] + bcls_ref[...]
    yctx = y[:, _D:] + bctx_ref[...]

    def ln_relu(x, g, b):
        mu = jnp.mean(x, axis=-1, keepdims=True)
        xc = x - mu
        var = jnp.mean(xc * xc, axis=-1, keepdims=True)
        return jnp.maximum(xc * jax.lax.rsqrt(var + _EPS) * g + b, 0.0)

    cls_e = ln_relu(ycls, gcls_ref[...], pcls_ref[...])
    ctx_e = ln_relu(yctx, gctx_ref[...], pctx_ref[...])

    bt = ids.shape[0]
    oh = (jax.lax.broadcasted_iota(jnp.int32, (bt, _NUM_SPECIAL), 1)
          == (ids - _SPECIAL_OFFSET)).astype(jnp.float32)
    out = jnp.dot(oh, tab_ref[...], preferred_element_type=jnp.float32)
    out = out + jnp.where(ids == _SPECIAL_OFFSET, cls_e, 0.0)
    out = out + jnp.where(ids == _SPECIAL_OFFSET + 1, ctx_e, 0.0)
    out_ref[...] = out


def kernel(token_ids, context_features, special_table, W_cls, b_cls, g_cls,
           beta_cls, W_ctx, b_ctx, g_ctx, beta_ctx):
    B, S = token_ids.shape
    T = B * S
    BT = 2048
    assert T % BT == 0
    NB = T // BT

    ids = token_ids.astype(jnp.int32).reshape(NB, BT, 1)
    cf = context_features.reshape(T, _NCTX)
    w_cls_pad = jnp.zeros((_NCTX, _D), jnp.float32).at[:W_cls.shape[0]].set(W_cls)
    wcat = jnp.concatenate([w_cls_pad, W_ctx], axis=1)

    row = lambda v: v.reshape(1, _D)
    full = lambda shape: pl.BlockSpec(shape, lambda i: (0,) * len(shape))

    out = pl.pallas_call(
        _body,
        grid=(NB,),
        in_specs=[
            pl.BlockSpec((1, BT, 1), lambda i: (i, 0, 0)),
            pl.BlockSpec((BT, _NCTX), lambda i: (i, 0)),
            full((_NUM_SPECIAL, _D)),
            full((_NCTX, 2 * _D)),
            full((1, _D)), full((1, _D)), full((1, _D)),
            full((1, _D)), full((1, _D)), full((1, _D)),
        ],
        out_specs=pl.BlockSpec((BT, _D), lambda i: (i, 0)),
        out_shape=jax.ShapeDtypeStruct((T, _D), jnp.float32),
        compiler_params=pltpu.CompilerParams(
            dimension_semantics=("arbitrary",),
        ),
    )(ids, cf, special_table, wcat,
      row(b_cls), row(g_cls), row(beta_cls),
      row(b_ctx), row(g_ctx), row(beta_ctx))
    return out.reshape(B, S, _D)
